# baseline (speedup 1.0000x reference)
import functools

import jax
import jax.numpy as jnp
from jax import lax
from jax.experimental import pallas as pl
from jax.experimental.pallas import tpu as pltpu
from jax.experimental.pallas import tpu_sc as plsc

_NUM_CORES = 2
_NUM_SUBCORES = 16
_NUM_WORKERS = _NUM_CORES * _NUM_SUBCORES
_CHUNK = 128

_BATCH = 16384
_DIM = 64
_B_PER_W = _BATCH // _NUM_WORKERS
_N_CHUNKS = _B_PER_W // _CHUNK


def _gather_body(idx_hbm, table_hbm, out_hbm, idx_v, rows_v, sem):
    wid = lax.axis_index("s") * _NUM_CORES + lax.axis_index("c")
    pltpu.sync_copy(idx_hbm.at[wid], idx_v)
    copies = [
        pltpu.async_copy(table_hbm.at[idx_v.at[j]], rows_v.at[j], sem)
        for j in range(_N_CHUNKS)
    ]
    for c in copies:
        c.wait()
    pltpu.sync_copy(rows_v, out_hbm.at[wid])


@jax.jit
def _sc_gather(idx, table):
    mesh = plsc.VectorSubcoreMesh(core_axis_name="c", subcore_axis_name="s")
    call = functools.partial(
        pl.kernel,
        mesh=mesh,
        out_type=jax.ShapeDtypeStruct(
            (_NUM_WORKERS, _N_CHUNKS, _CHUNK, _DIM), jnp.float32
        ),
        scratch_types=[
            pltpu.VMEM((_N_CHUNKS, _CHUNK), jnp.int32),
            pltpu.VMEM((_N_CHUNKS, _CHUNK, _DIM), jnp.float32),
            pltpu.SemaphoreType.DMA,
        ],
        compiler_params=pltpu.CompilerParams(use_tc_tiling_on_sc=False),
    )(_gather_body)
    return call(idx, table)


def kernel(indices, table):
    idx = indices.astype(jnp.int32).reshape(_NUM_WORKERS, _N_CHUNKS, _CHUNK)
    out = _sc_gather(idx, table)
    return out.reshape(_BATCH, _DIM)
